# baseline (speedup 1.0000x reference)
import functools

import jax
import jax.numpy as jnp
from jax import lax
from jax.experimental import pallas as pl
from jax.experimental.pallas import tpu as pltpu
from jax.experimental.pallas import tpu_sc as plsc

VOCAB = 1000
EMBED_DIM = 4000
PAD_DIM = 4096
BATCH = 4096
P = 4
BP = BATCH // P

_info = plsc.get_sparse_core_info()
_NC, _NS = _info.num_cores, _info.num_subcores
NW = _NC * _NS
BPW = BP // NW
CH = 8
NCHUNK = BPW // CH
NBUF = 3


def _body(idx_hbm, table_hbm, out_hbm, idx_v, *bufs_and_sems):
    bufs = bufs_and_sems[:NBUF]
    gsems = bufs_and_sems[NBUF:2 * NBUF]
    osems = bufs_and_sems[2 * NBUF:3 * NBUF]

    wid = lax.axis_index("s") * _NC + lax.axis_index("c")
    base = wid * BPW

    pltpu.sync_copy(idx_hbm.at[pl.ds(base, BPW)], idx_v)

    def gather(c, s):
        return pltpu.async_copy(
            table_hbm.at[idx_v.at[pl.ds(c * CH, CH)]], bufs[s], gsems[s])

    nbuf = min(NBUF, NCHUNK)
    gc = [gather(s, s) for s in range(nbuf)]
    oc = [None] * nbuf
    for c in range(NCHUNK):
        s = c % nbuf
        gc[s].wait()
        oc[s] = pltpu.async_copy(
            bufs[s], out_hbm.at[pl.ds(base + c * CH, CH)], osems[s])
        nxt = c + nbuf
        if nxt < NCHUNK:
            oc[s].wait()
            gc[s] = gather(nxt, s)
    for s in range(nbuf):
        oc[s].wait()


def _make_call():
    mesh = plsc.VectorSubcoreMesh(core_axis_name="c", subcore_axis_name="s")
    return functools.partial(
        pl.kernel,
        mesh=mesh,
        out_type=jax.ShapeDtypeStruct((BP, PAD_DIM), jnp.float32),
        scratch_types=(
            [pltpu.VMEM((BPW,), jnp.int32)]
            + [pltpu.VMEM((CH, PAD_DIM), jnp.float32)] * NBUF
            + [pltpu.SemaphoreType.DMA] * (2 * NBUF)
        ),
    )(_body)


_gather_call = _make_call()


def kernel(tokens, embedding_table):
    idx = tokens.astype(jnp.int32)
    table_p = jnp.pad(embedding_table, ((0, 0), (0, PAD_DIM - EMBED_DIM)))
    out = jnp.zeros((BATCH, EMBED_DIM), jnp.float32)
    for p in range(P):
        part = _gather_call(lax.slice(idx, (p * BP,), ((p + 1) * BP,)),
                            table_p)
        out = lax.dynamic_update_slice(out, part[:, :EMBED_DIM], (p * BP, 0))
    return out


# speedup vs baseline: 1.4501x; 1.4501x over previous
import functools

import jax
import jax.numpy as jnp
from jax import lax
from jax.experimental import pallas as pl
from jax.experimental.pallas import tpu as pltpu
from jax.experimental.pallas import tpu_sc as plsc

VOCAB = 1000
EMBED_DIM = 4000
PAD_DIM = 4096
MAIN = 3968
TAIL = EMBED_DIM - MAIN
BATCH = 4096

_info = plsc.get_sparse_core_info()
_NC, _NS = _info.num_cores, _info.num_subcores
NW = _NC * _NS
BPW = BATCH // NW
CH = 8
NCHUNK = BPW // CH
NBUF = 3


def _body(idx_hbm, table_hbm, out_hbm, tail_hbm, idx_v, *bufs_and_sems):
    bufs = bufs_and_sems[:NBUF]
    gsems = bufs_and_sems[NBUF:2 * NBUF]
    osems = bufs_and_sems[2 * NBUF:3 * NBUF]
    tsems = bufs_and_sems[3 * NBUF:4 * NBUF]

    wid = lax.axis_index("s") * _NC + lax.axis_index("c")
    base = wid * BPW

    pltpu.sync_copy(idx_hbm.at[pl.ds(base, BPW)], idx_v)

    def gather(c, s):
        return pltpu.async_copy(
            table_hbm.at[idx_v.at[pl.ds(c * CH, CH)]], bufs[s], gsems[s])

    gc = [gather(s, s) for s in range(NBUF)]
    oc = [None] * NBUF
    tc = [None] * NBUF
    for c in range(NCHUNK):
        s = c % NBUF
        rows = pl.ds(base + c * CH, CH)
        gc[s].wait()
        oc[s] = pltpu.async_copy(
            bufs[s].at[:, pl.ds(0, MAIN)], out_hbm.at[rows, pl.ds(0, MAIN)],
            osems[s])
        tc[s] = pltpu.async_copy(
            bufs[s].at[:, pl.ds(MAIN, 128)], tail_hbm.at[rows], tsems[s])
        nxt = c + NBUF
        if nxt < NCHUNK:
            oc[s].wait()
            tc[s].wait()
            gc[s] = gather(nxt, s)
    for s in range(NBUF):
        oc[s].wait()
        tc[s].wait()


def _make_call():
    mesh = plsc.VectorSubcoreMesh(core_axis_name="c", subcore_axis_name="s")
    return functools.partial(
        pl.kernel,
        mesh=mesh,
        out_type=[
            jax.ShapeDtypeStruct((BATCH, EMBED_DIM), jnp.float32),
            jax.ShapeDtypeStruct((BATCH, 128), jnp.float32),
        ],
        scratch_types=(
            [pltpu.VMEM((BPW,), jnp.int32)]
            + [pltpu.VMEM((CH, PAD_DIM), jnp.float32)] * NBUF
            + [pltpu.SemaphoreType.DMA] * (3 * NBUF)
        ),
    )(_body)


_gather_call = _make_call()


def kernel(tokens, embedding_table):
    idx = tokens.astype(jnp.int32)
    table_p = jnp.pad(embedding_table, ((0, 0), (0, PAD_DIM - EMBED_DIM)))
    out, tail = _gather_call(idx, table_p)
    return lax.dynamic_update_slice(out, tail[:, :TAIL], (0, MAIN))
